# Initial kernel scaffold; baseline (speedup 1.0000x reference)
import jax
import jax.numpy as jnp
from jax.experimental import pallas as pl


def kernel(x, edge_index, W_l, b_l, W_r):
    raise NotImplementedError("write your pallas kernel here")


# speedup vs baseline: 10.5440x; 10.5440x over previous
import functools

import jax
import jax.numpy as jnp
from jax import lax
from jax.experimental import pallas as pl
from jax.experimental.pallas import tpu as pltpu
from jax.experimental.pallas import tpu_sc as plsc

NC = 2
NS = 16
NW = NC * NS
LANES = 16
K = 80


def _sc_segment_sum(T, N, D, E, C):
    DR_TILES = 10
    RT = N // DR_TILES
    RZ = 8
    assert RT % RZ == 0
    NZ = RT // RZ
    CNT_R = N // DR_TILES

    mesh = plsc.VectorSubcoreMesh(
        core_axis_name="c", subcore_axis_name="s",
        num_cores=NC, num_subcores=NS)

    @functools.partial(
        pl.kernel,
        out_type=[
            jax.ShapeDtypeStruct((NC, T, N, D), jnp.float32),
            jax.ShapeDtypeStruct((NC, DR_TILES, 1, CNT_R), jnp.float32),
        ],
        mesh=mesh,
        scratch_types=[
            pltpu.VMEM((C, K), jnp.int32),
            pltpu.VMEM((C, K), jnp.int32),
            pltpu.VMEM((K, D), jnp.float32),
            pltpu.VMEM((K,), jnp.float32),
            pltpu.VMEM((RZ, D), jnp.float32),
            pltpu.VMEM((RZ, D), jnp.float32),
            pltpu.VMEM((1, CNT_R), jnp.float32),
            pltpu.VMEM_SHARED((N, D), jnp.float32),
            pltpu.VMEM_SHARED((N,), jnp.float32),
            pltpu.SemaphoreType.DMA,
        ],
    )
    def sc_kernel(src_hbm, dst_hbm, x_hbm, osum_hbm, ocnt_hbm,
                  src_v, dst_v, rows_v, ones_v, zrow_v, drn_v, cbuf_v,
                  acc_s, cnt_s, sem):
        c = lax.axis_index("c")
        s = lax.axis_index("s")
        wid = s * NC + c

        pltpu.sync_copy(src_hbm.at[wid], src_v)
        pltpu.sync_copy(dst_hbm.at[wid], dst_v)

        ones16 = jnp.ones((LANES,), jnp.float32)
        zero16 = jnp.zeros((LANES,), jnp.float32)

        def init_ones(i, carry):
            ones_v[pl.ds(i * LANES, LANES)] = ones16
            return carry
        lax.fori_loop(0, K // LANES, init_ones, 0)

        def init_zrow(i, carry):
            j = i // (D // LANES)
            k2 = (i % (D // LANES)) * LANES
            zrow_v[j, pl.ds(k2, LANES)] = zero16
            return carry
        lax.fori_loop(0, RZ * (D // LANES), init_zrow, 0)

        def init_cbuf(i, carry):
            cbuf_v[0, pl.ds(i * LANES, LANES)] = zero16
            return carry
        lax.fori_loop(0, CNT_R // LANES, init_cbuf, 0)

        @pl.when(s < DR_TILES)
        def _zero_cnt():
            pltpu.sync_copy(cbuf_v.at[0], cnt_s.at[pl.ds(s * CNT_R, CNT_R)])

        plsc.subcore_barrier()

        def cnt_chunk(j, carry):
            pltpu.sync_copy(ones_v, cnt_s.at[dst_v.at[j]], add=True)
            return carry
        lax.fori_loop(0, C, cnt_chunk, 0)

        plsc.subcore_barrier()

        @pl.when(s < DR_TILES)
        def _drain_cnt():
            pltpu.sync_copy(cnt_s.at[pl.ds(s * CNT_R, CNT_R)], cbuf_v.at[0])
            pltpu.sync_copy(cbuf_v, ocnt_hbm.at[c, s])

        def t_body(t, carry):
            @pl.when(s < DR_TILES)
            def _zero_acc():
                for z in range(NZ):
                    pltpu.sync_copy(zrow_v,
                                    acc_s.at[pl.ds(s * RT + z * RZ, RZ)])
            plsc.subcore_barrier()

            def chunk(j, carry2):
                pltpu.async_copy(x_hbm.at[src_v.at[j]], rows_v, sem).wait()
                pltpu.sync_copy(rows_v, acc_s.at[dst_v.at[j]], add=True)
                return carry2
            lax.fori_loop(0, C, chunk, 0)

            plsc.subcore_barrier()

            @pl.when(s < DR_TILES)
            def _drain_acc():
                for z in range(NZ):
                    r0 = s * RT + z * RZ
                    pltpu.sync_copy(acc_s.at[pl.ds(r0, RZ)], drn_v)
                    pltpu.sync_copy(drn_v, osum_hbm.at[c, t, pl.ds(r0, RZ)])

            def upd(i, carry2):
                j = i // (K // LANES)
                k2 = (i % (K // LANES)) * LANES
                src_v[j, pl.ds(k2, LANES)] = (
                    src_v[j, pl.ds(k2, LANES)] + jnp.int32(N))
                return carry2
            lax.fori_loop(0, C * (K // LANES), upd, 0)
            return carry

        lax.fori_loop(0, T, t_body, 0)

    return sc_kernel


def _tc_finish(T, N, D, BN):
    grid = (T, N // BN)

    def body(cnt_ref, p_ref, x_ref, wl_ref, bl_ref, wr_ref, o_ref):
        cnt = cnt_ref[:, 0] + cnt_ref[:, 1]
        ssum = p_ref[0, 0] + p_ref[1, 0]
        mean = ssum / jnp.clip(cnt, 1.0, None)[:, None]
        h = (jnp.dot(mean, wl_ref[...], preferred_element_type=jnp.float32)
             + jnp.dot(x_ref[0], wr_ref[...], preferred_element_type=jnp.float32)
             + bl_ref[0][None, :])
        o_ref[0] = h * 0.5 * (1.0 + lax.erf(h * 0.7071067811865476))

    return pl.pallas_call(
        body,
        grid=grid,
        in_specs=[
            pl.BlockSpec((BN, NC), lambda t, n: (n, 0)),
            pl.BlockSpec((NC, 1, BN, D), lambda t, n: (0, t, n, 0)),
            pl.BlockSpec((1, BN, D), lambda t, n: (t, n, 0)),
            pl.BlockSpec((D, D), lambda t, n: (0, 0)),
            pl.BlockSpec((1, D), lambda t, n: (0, 0)),
            pl.BlockSpec((D, D), lambda t, n: (0, 0)),
        ],
        out_specs=pl.BlockSpec((1, BN, D), lambda t, n: (t, n, 0)),
        out_shape=jax.ShapeDtypeStruct((T, N, D), jnp.float32),
    )


@jax.jit
def kernel(x, edge_index, W_l, b_l, W_r):
    T, N, D = x.shape
    E = edge_index.shape[1]
    assert E % NW == 0 and (E // NW) % K == 0
    C = E // NW // K

    src = edge_index[0].reshape(NW, C, K)
    dst = edge_index[1].reshape(NW, C, K)
    x_flat = x.reshape(T * N, D)

    osum, ocnt = _sc_segment_sum(T, N, D, E, C)(src, dst, x_flat)
    out = _tc_finish(T, N, D, 2000)(ocnt.reshape(NC, N).T, osum, x, W_l,
                                    b_l.reshape(1, D), W_r)
    return out


# speedup vs baseline: 14.1104x; 1.3382x over previous
import functools

import jax
import jax.numpy as jnp
from jax import lax
from jax.experimental import pallas as pl
from jax.experimental.pallas import tpu as pltpu
from jax.experimental.pallas import tpu_sc as plsc

NC = 2
NS = 16
NW = NC * NS
LANES = 16
K = 40
SB = 50


def _sc_segment_sum(T, N, D, E, C):
    DR_TILES = 10
    RT = N // DR_TILES
    CNT_R = N // DR_TILES
    NB = C // SB
    assert C % SB == 0 and SB % 2 == 0
    KP = -(-K // LANES) * LANES

    mesh = plsc.VectorSubcoreMesh(
        core_axis_name="c", subcore_axis_name="s",
        num_cores=NC, num_subcores=NS)

    @functools.partial(
        pl.kernel,
        out_type=[
            jax.ShapeDtypeStruct((NC, T, N, D), jnp.float32),
            jax.ShapeDtypeStruct((NC, DR_TILES, 1, CNT_R), jnp.float32),
        ],
        mesh=mesh,
        scratch_types=[
            pltpu.VMEM((SB, K), jnp.int32),
            pltpu.VMEM((SB, K), jnp.int32),
            pltpu.VMEM((K, D), jnp.float32),
            pltpu.VMEM((K, D), jnp.float32),
            pltpu.VMEM((KP,), jnp.float32),
            pltpu.VMEM((1, N // 10), jnp.float32),
            pltpu.VMEM_SHARED((N, D), jnp.float32),
            pltpu.VMEM_SHARED((N,), jnp.float32),
            pltpu.SemaphoreType.DMA,
            pltpu.SemaphoreType.DMA,
        ],
    )
    def sc_kernel(srcx_hbm, dst_hbm, x_hbm, z2d_hbm,
                  osum_hbm, ocnt_hbm,
                  src_v, dst_v, rb0, rb1, ones_v, cbuf_v,
                  acc_s, cnt_s, sem0, sem1):
        c = lax.axis_index("c")
        s = lax.axis_index("s")
        wid = s * NC + c

        ones16 = jnp.ones((LANES,), jnp.float32)
        zero16 = jnp.zeros((LANES,), jnp.float32)

        def init_ones(i, carry):
            ones_v[pl.ds(i * LANES, LANES)] = ones16
            return carry
        lax.fori_loop(0, KP // LANES, init_ones, 0)

        def init_cbuf(i, carry):
            cbuf_v[0, pl.ds(i * LANES, LANES)] = zero16
            return carry
        lax.fori_loop(0, CNT_R // LANES, init_cbuf, 0)

        @pl.when(s < DR_TILES)
        def _zero_cnt():
            pltpu.sync_copy(cbuf_v.at[0], cnt_s.at[pl.ds(s * CNT_R, CNT_R)])

        plsc.subcore_barrier()

        def cnt_block(b, carry):
            pltpu.sync_copy(dst_hbm.at[wid * NB + b], dst_v)

            def cnt_chunk(j, carry2):
                pltpu.sync_copy(ones_v.at[pl.ds(0, K)],
                                cnt_s.at[dst_v.at[j]], add=True)
                return carry2
            lax.fori_loop(0, SB, cnt_chunk, 0)
            return carry
        lax.fori_loop(0, NB, cnt_block, 0)

        plsc.subcore_barrier()

        @pl.when(s < DR_TILES)
        def _drain_cnt():
            pltpu.sync_copy(cnt_s.at[pl.ds(s * CNT_R, CNT_R)], cbuf_v.at[0])
            pltpu.sync_copy(cbuf_v, ocnt_hbm.at[c, s])

        def t_body(t, carry):
            @pl.when(s < DR_TILES)
            def _zero_acc():
                pltpu.sync_copy(z2d_hbm, acc_s.at[pl.ds(s * RT, RT)])
            plsc.subcore_barrier()

            def block(b, carry2):
                g = (t * NW + wid) * NB + b
                pltpu.sync_copy(srcx_hbm.at[g], src_v)
                pltpu.sync_copy(dst_hbm.at[wid * NB + b], dst_v)

                pltpu.async_copy(x_hbm.at[src_v.at[0]], rb0, sem0)

                def pair(i, carry3):
                    j = 2 * i
                    pltpu.async_copy(x_hbm.at[src_v.at[j + 1]], rb1, sem1)
                    pltpu.make_async_copy(
                        x_hbm.at[pl.ds(0, K)], rb0, sem0).wait()
                    pltpu.sync_copy(rb0, acc_s.at[dst_v.at[j]], add=True)
                    pltpu.async_copy(x_hbm.at[src_v.at[j + 2]], rb0, sem0)
                    pltpu.make_async_copy(
                        x_hbm.at[pl.ds(0, K)], rb1, sem1).wait()
                    pltpu.sync_copy(rb1, acc_s.at[dst_v.at[j + 1]],
                                    add=True)
                    return carry3
                lax.fori_loop(0, SB // 2 - 1, pair, 0)

                pltpu.async_copy(x_hbm.at[src_v.at[SB - 1]], rb1, sem1)
                pltpu.make_async_copy(
                    x_hbm.at[pl.ds(0, K)], rb0, sem0).wait()
                pltpu.sync_copy(rb0, acc_s.at[dst_v.at[SB - 2]], add=True)
                pltpu.make_async_copy(
                    x_hbm.at[pl.ds(0, K)], rb1, sem1).wait()
                pltpu.sync_copy(rb1, acc_s.at[dst_v.at[SB - 1]], add=True)
                return carry2
            lax.fori_loop(0, NB, block, 0)

            plsc.subcore_barrier()

            @pl.when(s < DR_TILES)
            def _drain_acc():
                r0 = s * RT
                pltpu.sync_copy(acc_s.at[pl.ds(r0, RT)],
                                osum_hbm.at[c, t, pl.ds(r0, RT)])
            return carry

        lax.fori_loop(0, T, t_body, 0)

    return sc_kernel


def _tc_finish(T, N, D, BN):
    grid = (T, N // BN)

    def body(cnt_ref, p_ref, x_ref, wl_ref, bl_ref, wr_ref, o_ref):
        cnt = cnt_ref[:, 0] + cnt_ref[:, 1]
        ssum = p_ref[0, 0] + p_ref[1, 0]
        mean = ssum / jnp.clip(cnt, 1.0, None)[:, None]
        h = (jnp.dot(mean, wl_ref[...], preferred_element_type=jnp.float32)
             + jnp.dot(x_ref[0], wr_ref[...],
                       preferred_element_type=jnp.float32)
             + bl_ref[0][None, :])
        o_ref[0] = h * 0.5 * (1.0 + lax.erf(h * 0.7071067811865476))

    return pl.pallas_call(
        body,
        grid=grid,
        in_specs=[
            pl.BlockSpec((BN, NC), lambda t, n: (n, 0)),
            pl.BlockSpec((NC, 1, BN, D), lambda t, n: (0, t, n, 0)),
            pl.BlockSpec((1, BN, D), lambda t, n: (t, n, 0)),
            pl.BlockSpec((D, D), lambda t, n: (0, 0)),
            pl.BlockSpec((1, D), lambda t, n: (0, 0)),
            pl.BlockSpec((D, D), lambda t, n: (0, 0)),
        ],
        out_specs=pl.BlockSpec((1, BN, D), lambda t, n: (t, n, 0)),
        out_shape=jax.ShapeDtypeStruct((T, N, D), jnp.float32),
    )


@jax.jit
def kernel(x, edge_index, W_l, b_l, W_r):
    T, N, D = x.shape
    E = edge_index.shape[1]
    assert E % NW == 0 and (E // NW) % K == 0
    C = E // NW // K
    NB = C // SB

    src = edge_index[0].reshape(1, NW, NB, SB, K)
    offs = (jnp.arange(T, dtype=jnp.int32) * N).reshape(T, 1, 1, 1, 1)
    srcx = (src + offs).reshape(T * NW * NB, SB, K)
    dst = edge_index[1].reshape(NW * NB, SB, K)
    x_flat = x.reshape(T * N, D)
    RT = N // 10
    z2d = jnp.zeros((RT, D), jnp.float32)

    osum, ocnt = _sc_segment_sum(T, N, D, E, C)(srcx, dst, x_flat, z2d)
    out = _tc_finish(T, N, D, 2000)(ocnt.reshape(NC, N).T, osum, x, W_l,
                                    b_l.reshape(1, D), W_r)
    return out
